# baseline (speedup 1.0000x reference)
import numpy as np

import jax
import jax.numpy as jnp
from jax import lax
from jax.experimental import pallas as pl
from jax.experimental.pallas import tpu as pltpu
from jax.experimental.pallas import tpu_sc as plsc

N = 10000
E = 320000
D = 128

NC = 2
NS = 16
LANES = 16
STAGE = 80
N_STAGES = E // STAGE
STAGES_PER_SC = N_STAGES // NC
TILE_STAGES = STAGES_PER_SC // NS
BATCH = 25
MAIN = (TILE_STAGES - 5) // 6 * 6
SEXTETS = MAIN // 6
UNROLL = 4
NODE_BASE = 624
NODE_EXTRA_TILES = (N - NS * NODE_BASE) // 8

_PERM = np.empty((D,), np.int32)
for _k in range(D // 32):
    for _j in range(16):
        _PERM[32 * _k + 2 * _j] = 32 * _k + _j
        _PERM[32 * _k + 2 * _j + 1] = 32 * _k + 16 + _j


def _linear_body(x_ref, w_ref, b_ref, o_ref):
    o_ref[...] = (lax.dot_general(
        x_ref[...], w_ref[...],
        dimension_numbers=(((1,), (1,)), ((), ())),
        preferred_element_type=jnp.float32,
    ) + b_ref[...]).astype(jnp.bfloat16)


def _linear(x, w, b2):
    blk = 1000
    return pl.pallas_call(
        _linear_body,
        out_shape=jax.ShapeDtypeStruct((N, D), jnp.bfloat16),
        grid=(N // blk,),
        in_specs=[
            pl.BlockSpec((blk, D), lambda i: (i, 0)),
            pl.BlockSpec((D, D), lambda i: (0, 0)),
            pl.BlockSpec((1, D), lambda i: (0, 0)),
        ],
        out_specs=pl.BlockSpec((blk, D), lambda i: (i, 0)),
    )(x, w, b2)


def _add_body(a_ref, b_ref, o_ref):
    o_ref[...] = a_ref[...] + b_ref[...]


def _combine(p0, p1):
    blk = 1000
    return pl.pallas_call(
        _add_body,
        out_shape=jax.ShapeDtypeStruct((N, D), jnp.float32),
        grid=(N // blk,),
        in_specs=[
            pl.BlockSpec((blk, D), lambda i: (i, 0)),
            pl.BlockSpec((blk, D), lambda i: (i, 0)),
        ],
        out_specs=pl.BlockSpec((blk, D), lambda i: (i, 0)),
    )(p0, p1)


def _sc_body(hidden32, src3, dst3, w, zrows, out,
             sidx, didx, wbuf, g0, g1, g2, f0, f1, acc,
             gsem0, gsem1, gsem2, ssem0, ssem1):
    c = lax.axis_index("c")
    s = lax.axis_index("s")
    gbuf = (g0, g1, g2)
    fbuf = (f0, f1)
    gsem = (gsem0, gsem1, gsem2)
    ssem = (ssem0, ssem1)

    node_start = (s * (NODE_BASE // 8) + jnp.minimum(s, NODE_EXTRA_TILES)) * 8

    pltpu.sync_copy(zrows.at[pl.ds(0, NODE_BASE)],
                    acc.at[pl.ds(node_start, NODE_BASE)])

    @pl.when(s < NODE_EXTRA_TILES)
    def _():
        pltpu.sync_copy(zrows.at[pl.ds(NODE_BASE, 8)],
                        acc.at[pl.ds(node_start + NODE_BASE, 8)])

    row0 = c * STAGES_PER_SC + s * TILE_STAGES

    def stage_batch(j, half):
        grow = row0 + j * BATCH
        pltpu.sync_copy(src3.at[pl.ds(grow, BATCH)], sidx.at[half])
        pltpu.sync_copy(dst3.at[pl.ds(grow, BATCH)], didx.at[half])
        pltpu.sync_copy(w.at[pl.ds(grow * STAGE, BATCH * STAGE)],
                        wbuf.at[pl.ds(half * BATCH * STAGE, BATCH * STAGE)])

    stage_batch(0, 0)
    plsc.subcore_barrier()

    def gather_start(i, gb):
        h = (i // BATCH) % 2
        lr = i % BATCH
        pltpu.async_copy(hidden32.at[sidx.at[h, lr, 0]], gbuf[gb], gsem[gb])

    def gather_wait(gb):
        pltpu.make_async_copy(hidden32.at[sidx.at[0, 0, 0]], gbuf[gb],
                              gsem[gb]).wait()

    def scatter_start(i, fb):
        h = (i // BATCH) % 2
        lr = i % BATCH
        pltpu.async_copy(fbuf[fb], acc.at[didx.at[h, lr, 0]], ssem[fb],
                         add=True)

    def scatter_wait(fb):
        pltpu.make_async_copy(fbuf[fb], acc.at[didx.at[0, 0, 0]],
                              ssem[fb]).wait()

    def scale(i, gb, fb):
        h = (i // BATCH) % 2
        base = h * (BATCH * STAGE) + (i % BATCH) * STAGE
        gr, fr = gbuf[gb], fbuf[fb]

        def edge_body(e4, carry):
            for u in range(UNROLL):
                e = e4 * UNROLL + u
                wb = plsc.load_gather(
                    wbuf, [jnp.full((LANES,), base + e, jnp.int32)])
                for k in range(D // 32):
                    packed = gr[e, pl.ds(k * 16, 16)]
                    bf = plsc.bitcast(packed, jnp.bfloat16)
                    lo, hi = plsc.unpack(
                        bf, format=plsc.PackFormat.INTERLEAVED)
                    fr[e, pl.ds(k * 32, 16)] = lo * wb
                    fr[e, pl.ds(k * 32 + 16, 16)] = hi * wb
            return carry

        lax.fori_loop(0, STAGE // UNROLL, edge_body, 0)

    def do_stage(i, gb, fb, *, in_loop=True):
        gather_wait(gb)
        scatter_wait(fb)
        if in_loop:
            nxt = i + 2

            @pl.when(nxt % BATCH == 0)
            def _():
                stage_batch(nxt // BATCH, (nxt // BATCH) % 2)

            gather_start(nxt, (gb + 2) % 3)
        scale(i, gb, fb)
        scatter_start(i, fb)

    gather_start(0, 0)
    gather_start(1, 1)

    def sextet_body(t, carry):
        i0 = 6 * t
        for u in range(6):
            if u < 2:
                @pl.when(t > 0)
                def _():
                    scatter_wait(u % 2)
                gather_wait(u % 3)
                nxt = i0 + u + 2

                @pl.when(nxt % BATCH == 0)
                def _():
                    stage_batch(nxt // BATCH, (nxt // BATCH) % 2)

                gather_start(nxt, (u + 2) % 3)
                scale(i0 + u, u % 3, u % 2)
                scatter_start(i0 + u, u % 2)
            else:
                do_stage(i0 + u, u % 3, u % 2)
        return carry

    lax.fori_loop(0, SEXTETS, sextet_body, 0)

    for i in range(MAIN, TILE_STAGES):
        do_stage(i, i % 3, i % 2, in_loop=(i + 2 < TILE_STAGES))

    scatter_wait((TILE_STAGES - 2) % 2)
    scatter_wait((TILE_STAGES - 1) % 2)

    plsc.subcore_barrier()
    pltpu.sync_copy(acc.at[pl.ds(node_start, NODE_BASE)],
                    out.at[c, pl.ds(node_start, NODE_BASE)])

    @pl.when(s < NODE_EXTRA_TILES)
    def _():
        pltpu.sync_copy(acc.at[pl.ds(node_start + NODE_BASE, 8)],
                        out.at[c, pl.ds(node_start + NODE_BASE, 8)])


_sc_scatter = pl.kernel(
    _sc_body,
    out_type=jax.ShapeDtypeStruct((NC, N, D), jnp.float32),
    mesh=plsc.VectorSubcoreMesh(core_axis_name="c", subcore_axis_name="s"),
    compiler_params=pltpu.CompilerParams(needs_layout_passes=False,
                                         use_tc_tiling_on_sc=False),
    scratch_types=[
        pltpu.VMEM((2, BATCH, 1, STAGE), jnp.int32),
        pltpu.VMEM((2, BATCH, 1, STAGE), jnp.int32),
        pltpu.VMEM((2 * BATCH * STAGE,), jnp.float32),
        pltpu.VMEM((STAGE, D // 2), jnp.int32),
        pltpu.VMEM((STAGE, D // 2), jnp.int32),
        pltpu.VMEM((STAGE, D // 2), jnp.int32),
        pltpu.VMEM((STAGE, D), jnp.float32),
        pltpu.VMEM((STAGE, D), jnp.float32),
        pltpu.VMEM_SHARED((N, D), jnp.float32),
        pltpu.SemaphoreType.DMA,
        pltpu.SemaphoreType.DMA,
        pltpu.SemaphoreType.DMA,
        pltpu.SemaphoreType.DMA,
        pltpu.SemaphoreType.DMA,
    ],
)


def kernel(ego_embedding, edge_index, edge_weight, W, b):
    perm = jnp.asarray(_PERM)
    hidden_bf = _linear(ego_embedding, W[perm], b[perm].reshape(1, D))
    hidden32 = lax.bitcast_convert_type(
        hidden_bf.reshape(N, D // 2, 2), jnp.int32)
    src3 = edge_index[0].reshape(N_STAGES, 1, STAGE)
    dst3 = edge_index[1].reshape(N_STAGES, 1, STAGE)
    zrows = jnp.zeros((NODE_BASE + 8, D), jnp.float32)
    partials = _sc_scatter(hidden32, src3, dst3, edge_weight, zrows)
    return _combine(partials[0], partials[1])


# speedup vs baseline: 1.9045x; 1.9045x over previous
import jax
import jax.numpy as jnp
from jax import lax
from jax.experimental import pallas as pl
from jax.experimental.pallas import tpu as pltpu
from jax.experimental.pallas import tpu_sc as plsc

N = 10000
E = 320000
D = 128

NC = 2
NS = 16
LANES = 16
STAGE = 80
SPLIT = 2
N_STAGES = E // STAGE
STAGES_PER_SC = N_STAGES // NC
TILE_STAGES = STAGES_PER_SC // NS
BATCH = 25
TRIPLES = (TILE_STAGES - 2) // 3
UNROLL = 4
NODE_BASE = 624
NODE_EXTRA_TILES = (N - NS * NODE_BASE) // 8


def _linear_body(x_ref, w_ref, b_ref, o_ref):
    o_ref[...] = lax.dot_general(
        x_ref[...], w_ref[...],
        dimension_numbers=(((1,), (1,)), ((), ())),
        preferred_element_type=jnp.float32,
    ) + b_ref[...]


def _linear(x, w, b2):
    blk = 1000
    return pl.pallas_call(
        _linear_body,
        out_shape=jax.ShapeDtypeStruct((N, D), jnp.float32),
        grid=(N // blk,),
        in_specs=[
            pl.BlockSpec((blk, D), lambda i: (i, 0)),
            pl.BlockSpec((D, D), lambda i: (0, 0)),
            pl.BlockSpec((1, D), lambda i: (0, 0)),
        ],
        out_specs=pl.BlockSpec((blk, D), lambda i: (i, 0)),
    )(x, w, b2)


def _add_body(a_ref, b_ref, o_ref):
    o_ref[...] = a_ref[...] + b_ref[...]


def _combine(p0, p1):
    blk = 1000
    return pl.pallas_call(
        _add_body,
        out_shape=jax.ShapeDtypeStruct((N, D), jnp.float32),
        grid=(N // blk,),
        in_specs=[
            pl.BlockSpec((blk, D), lambda i: (i, 0)),
            pl.BlockSpec((blk, D), lambda i: (i, 0)),
        ],
        out_specs=pl.BlockSpec((blk, D), lambda i: (i, 0)),
    )(p0, p1)


def _sc_body(hidden, src3, dst3, w, zrows, out,
             sidx, didx, wbuf, rows0, rows1, rows2, acc,
             gsem0, gsem1, gsem2, ssem0, ssem1, ssem2):
    c = lax.axis_index("c")
    s = lax.axis_index("s")
    rows = (rows0, rows1, rows2)
    gsem = (gsem0, gsem1, gsem2)
    ssem = (ssem0, ssem1, ssem2)

    node_start = (s * (NODE_BASE // 8) + jnp.minimum(s, NODE_EXTRA_TILES)) * 8

    pltpu.sync_copy(zrows.at[pl.ds(0, NODE_BASE)],
                    acc.at[pl.ds(node_start, NODE_BASE)])

    @pl.when(s < NODE_EXTRA_TILES)
    def _():
        pltpu.sync_copy(zrows.at[pl.ds(NODE_BASE, 8)],
                        acc.at[pl.ds(node_start + NODE_BASE, 8)])

    row0 = c * STAGES_PER_SC + s * TILE_STAGES

    def stage_batch(j, half):
        grow = row0 + j * BATCH
        pltpu.sync_copy(src3.at[pl.ds(grow, BATCH)], sidx.at[half])
        pltpu.sync_copy(dst3.at[pl.ds(grow, BATCH)], didx.at[half])
        pltpu.sync_copy(w.at[pl.ds(grow * STAGE, BATCH * STAGE)],
                        wbuf.at[pl.ds(half * BATCH * STAGE, BATCH * STAGE)])

    stage_batch(0, 0)
    plsc.subcore_barrier()

    SUB = STAGE // SPLIT

    def gather_start(i, buf, sem):
        h = (i // BATCH) % 2
        lr = i % BATCH
        for j in range(SPLIT):
            pltpu.async_copy(
                hidden.at[sidx.at[h, lr, 0, pl.ds(j * SUB, SUB)]],
                buf.at[pl.ds(j * SUB, SUB)], sem)

    def gather_wait(buf, sem):
        for j in range(SPLIT):
            pltpu.make_async_copy(hidden.at[sidx.at[0, 0, 0, pl.ds(0, SUB)]],
                                  buf.at[pl.ds(j * SUB, SUB)], sem).wait()

    def scatter_start(i, buf, sem):
        h = (i // BATCH) % 2
        lr = i % BATCH
        pltpu.async_copy(buf, acc.at[didx.at[h, lr, 0]], sem, add=True)

    def scatter_wait(buf, sem):
        pltpu.make_async_copy(buf, acc.at[didx.at[0, 0, 0]], sem).wait()

    def scale(buf, i):
        h = (i // BATCH) % 2
        base = h * (BATCH * STAGE) + (i % BATCH) * STAGE

        def edge_body(e4, carry):
            for u in range(UNROLL):
                e = e4 * UNROLL + u
                wb = plsc.load_gather(
                    wbuf, [jnp.full((LANES,), base + e, jnp.int32)])
                for k in range(D // LANES):
                    sl = pl.ds(k * LANES, LANES)
                    buf[e, sl] = buf[e, sl] * wb
            return carry

        lax.fori_loop(0, STAGE // UNROLL, edge_body, 0)

    gather_start(0, rows[0], gsem[0])
    gather_start(1, rows[1], gsem[1])

    def triple_body(t, carry):
        i0 = 3 * t
        for b in range(3):
            i = i0 + b
            buf, gs, cs = rows[b], gsem[b], ssem[b]
            gather_wait(buf, gs)
            scale(buf, i)
            if b == 0:
                @pl.when(t > 0)
                def _():
                    scatter_wait(rows[2], ssem[2])
            else:
                scatter_wait(rows[b - 1], ssem[b - 1])
            nxt = i + 2

            @pl.when(nxt % BATCH == 0)
            def _():
                stage_batch(nxt // BATCH, (nxt // BATCH) % 2)

            gather_start(nxt, rows[(b + 2) % 3], gsem[(b + 2) % 3])
            scatter_start(i, buf, cs)
        return carry

    lax.fori_loop(0, TRIPLES, triple_body, 0)

    gather_wait(rows[0], gsem[0])
    scale(rows[0], TILE_STAGES - 2)
    scatter_start(TILE_STAGES - 2, rows[0], ssem[0])
    gather_wait(rows[1], gsem[1])
    scale(rows[1], TILE_STAGES - 1)
    scatter_start(TILE_STAGES - 1, rows[1], ssem[1])
    scatter_wait(rows[2], ssem[2])
    scatter_wait(rows[0], ssem[0])
    scatter_wait(rows[1], ssem[1])

    plsc.subcore_barrier()
    pltpu.sync_copy(acc.at[pl.ds(node_start, NODE_BASE)],
                    out.at[c, pl.ds(node_start, NODE_BASE)])

    @pl.when(s < NODE_EXTRA_TILES)
    def _():
        pltpu.sync_copy(acc.at[pl.ds(node_start + NODE_BASE, 8)],
                        out.at[c, pl.ds(node_start + NODE_BASE, 8)])


_sc_scatter = pl.kernel(
    _sc_body,
    out_type=jax.ShapeDtypeStruct((NC, N, D), jnp.float32),
    mesh=plsc.VectorSubcoreMesh(core_axis_name="c", subcore_axis_name="s"),
    compiler_params=pltpu.CompilerParams(needs_layout_passes=False,
                                         skip_device_barrier=True),
    scratch_types=[
        pltpu.VMEM((2, BATCH, 1, STAGE), jnp.int32),
        pltpu.VMEM((2, BATCH, 1, STAGE), jnp.int32),
        pltpu.VMEM((2 * BATCH * STAGE,), jnp.float32),
        pltpu.VMEM((STAGE, D), jnp.float32),
        pltpu.VMEM((STAGE, D), jnp.float32),
        pltpu.VMEM((STAGE, D), jnp.float32),
        pltpu.VMEM_SHARED((N, D), jnp.float32),
        pltpu.SemaphoreType.DMA,
        pltpu.SemaphoreType.DMA,
        pltpu.SemaphoreType.DMA,
        pltpu.SemaphoreType.DMA,
        pltpu.SemaphoreType.DMA,
        pltpu.SemaphoreType.DMA,
    ],
)


def kernel(ego_embedding, edge_index, edge_weight, W, b):
    hidden = _linear(ego_embedding, W, b.reshape(1, D))
    src3 = edge_index[0].reshape(N_STAGES, 1, STAGE)
    dst3 = edge_index[1].reshape(N_STAGES, 1, STAGE)
    zrows = jnp.zeros((NODE_BASE + 8, D), jnp.float32)
    partials = _sc_scatter(hidden, src3, dst3, edge_weight, zrows)
    return _combine(partials[0], partials[1])


# speedup vs baseline: 1.9088x; 1.0023x over previous
import jax
import jax.numpy as jnp
from jax import lax
from jax.experimental import pallas as pl
from jax.experimental.pallas import tpu as pltpu
from jax.experimental.pallas import tpu_sc as plsc

N = 10000
E = 320000
D = 128

NC = 2
NS = 16
LANES = 16
STAGE = 80
SPLIT = 2
N_STAGES = E // STAGE
STAGES_PER_SC = N_STAGES // NC
TILE_STAGES = STAGES_PER_SC // NS
BATCH = 25
TRIPLES = (TILE_STAGES - 2) // 3
UNROLL = 4
NODE_BASE = 624
NODE_EXTRA_TILES = (N - NS * NODE_BASE) // 8


def _linear_body(x_ref, w_ref, b_ref, o_ref):
    o_ref[...] = lax.dot_general(
        x_ref[...], w_ref[...],
        dimension_numbers=(((1,), (1,)), ((), ())),
        preferred_element_type=jnp.float32,
    ) + b_ref[...]


def _linear(x, w, b2):
    blk = 1000
    return pl.pallas_call(
        _linear_body,
        out_shape=jax.ShapeDtypeStruct((N, D), jnp.float32),
        grid=(N // blk,),
        in_specs=[
            pl.BlockSpec((blk, D), lambda i: (i, 0)),
            pl.BlockSpec((D, D), lambda i: (0, 0)),
            pl.BlockSpec((1, D), lambda i: (0, 0)),
        ],
        out_specs=pl.BlockSpec((blk, D), lambda i: (i, 0)),
    )(x, w, b2)


def _add_body(a_ref, b_ref, o_ref):
    o_ref[...] = a_ref[...] + b_ref[...]


def _combine(p0, p1):
    blk = 1000
    return pl.pallas_call(
        _add_body,
        out_shape=jax.ShapeDtypeStruct((N, D), jnp.float32),
        grid=(N // blk,),
        in_specs=[
            pl.BlockSpec((blk, D), lambda i: (i, 0)),
            pl.BlockSpec((blk, D), lambda i: (i, 0)),
        ],
        out_specs=pl.BlockSpec((blk, D), lambda i: (i, 0)),
    )(p0, p1)


def _sc_body(hidden, src, dst3, w, zrows, out,
             sidx, didx, wbuf, rows0, rows1, rows2, acc,
             gsem0, gsem1, gsem2, ssem0, ssem1, ssem2):
    c = lax.axis_index("c")
    s = lax.axis_index("s")
    rows = (rows0, rows1, rows2)
    gsem = (gsem0, gsem1, gsem2)
    ssem = (ssem0, ssem1, ssem2)

    node_start = (s * (NODE_BASE // 8) + jnp.minimum(s, NODE_EXTRA_TILES)) * 8

    pltpu.sync_copy(zrows.at[pl.ds(0, NODE_BASE)],
                    acc.at[pl.ds(node_start, NODE_BASE)])

    @pl.when(s < NODE_EXTRA_TILES)
    def _():
        pltpu.sync_copy(zrows.at[pl.ds(NODE_BASE, 8)],
                        acc.at[pl.ds(node_start + NODE_BASE, 8)])

    row0 = c * STAGES_PER_SC + s * TILE_STAGES

    def stage_batch(j, half):
        grow = row0 + j * BATCH
        pltpu.sync_copy(src.at[pl.ds(grow * STAGE, BATCH * STAGE)],
                        sidx.at[pl.ds(half * BATCH * STAGE, BATCH * STAGE)])
        pltpu.sync_copy(dst3.at[pl.ds(grow, BATCH)], didx.at[half])
        pltpu.sync_copy(w.at[pl.ds(grow * STAGE, BATCH * STAGE)],
                        wbuf.at[pl.ds(half * BATCH * STAGE, BATCH * STAGE)])

    stage_batch(0, 0)
    plsc.subcore_barrier()

    SUB = STAGE // SPLIT

    def gather_start(i, buf, sem):
        h = (i // BATCH) % 2
        base = h * (BATCH * STAGE) + (i % BATCH) * STAGE
        for j in range(SPLIT):
            pltpu.async_copy(
                hidden.at[sidx.at[pl.ds(base + j * SUB, SUB)]],
                buf.at[pl.ds(j * SUB, SUB)], sem)

    def gather_wait(buf, sem):
        for j in range(SPLIT):
            pltpu.make_async_copy(hidden.at[sidx.at[pl.ds(0, SUB)]],
                                  buf.at[pl.ds(j * SUB, SUB)], sem).wait()

    def scatter_start(i, buf, sem):
        h = (i // BATCH) % 2
        lr = i % BATCH
        pltpu.async_copy(buf, acc.at[didx.at[h, lr, 0]], sem, add=True)

    def scatter_wait(buf, sem):
        pltpu.make_async_copy(buf, acc.at[didx.at[0, 0, 0]], sem).wait()

    def scale(buf, i):
        h = (i // BATCH) % 2
        base = h * (BATCH * STAGE) + (i % BATCH) * STAGE

        def edge_body(e4, carry):
            for u in range(UNROLL):
                e = e4 * UNROLL + u
                wb = plsc.load_gather(
                    wbuf, [jnp.full((LANES,), base + e, jnp.int32)])
                for k in range(D // LANES):
                    sl = pl.ds(k * LANES, LANES)
                    buf[e, sl] = buf[e, sl] * wb
            return carry

        lax.fori_loop(0, STAGE // UNROLL, edge_body, 0)

    gather_start(0, rows[0], gsem[0])
    gather_start(1, rows[1], gsem[1])

    def triple_body(t, carry):
        i0 = 3 * t
        for b in range(3):
            i = i0 + b
            buf, gs, cs = rows[b], gsem[b], ssem[b]
            gather_wait(buf, gs)
            scale(buf, i)
            if b == 0:
                @pl.when(t > 0)
                def _():
                    scatter_wait(rows[2], ssem[2])
            else:
                scatter_wait(rows[b - 1], ssem[b - 1])
            nxt = i + 2

            @pl.when(nxt % BATCH == 0)
            def _():
                stage_batch(nxt // BATCH, (nxt // BATCH) % 2)

            gather_start(nxt, rows[(b + 2) % 3], gsem[(b + 2) % 3])
            scatter_start(i, buf, cs)
        return carry

    lax.fori_loop(0, TRIPLES, triple_body, 0)

    gather_wait(rows[0], gsem[0])
    scale(rows[0], TILE_STAGES - 2)
    scatter_start(TILE_STAGES - 2, rows[0], ssem[0])
    gather_wait(rows[1], gsem[1])
    scale(rows[1], TILE_STAGES - 1)
    scatter_start(TILE_STAGES - 1, rows[1], ssem[1])
    scatter_wait(rows[2], ssem[2])
    scatter_wait(rows[0], ssem[0])
    scatter_wait(rows[1], ssem[1])

    plsc.subcore_barrier()
    pltpu.sync_copy(acc.at[pl.ds(node_start, NODE_BASE)],
                    out.at[c, pl.ds(node_start, NODE_BASE)])

    @pl.when(s < NODE_EXTRA_TILES)
    def _():
        pltpu.sync_copy(acc.at[pl.ds(node_start + NODE_BASE, 8)],
                        out.at[c, pl.ds(node_start + NODE_BASE, 8)])


_sc_scatter = pl.kernel(
    _sc_body,
    out_type=jax.ShapeDtypeStruct((NC, N, D), jnp.float32),
    mesh=plsc.VectorSubcoreMesh(core_axis_name="c", subcore_axis_name="s"),
    compiler_params=pltpu.CompilerParams(needs_layout_passes=False,
                                         skip_device_barrier=True),
    scratch_types=[
        pltpu.VMEM((2 * BATCH * STAGE,), jnp.int32),
        pltpu.VMEM((2, BATCH, 1, STAGE), jnp.int32),
        pltpu.VMEM((2 * BATCH * STAGE,), jnp.float32),
        pltpu.VMEM((STAGE, D), jnp.float32),
        pltpu.VMEM((STAGE, D), jnp.float32),
        pltpu.VMEM((STAGE, D), jnp.float32),
        pltpu.VMEM_SHARED((N, D), jnp.float32),
        pltpu.SemaphoreType.DMA,
        pltpu.SemaphoreType.DMA,
        pltpu.SemaphoreType.DMA,
        pltpu.SemaphoreType.DMA,
        pltpu.SemaphoreType.DMA,
        pltpu.SemaphoreType.DMA,
    ],
)


def kernel(ego_embedding, edge_index, edge_weight, W, b):
    hidden = _linear(ego_embedding, W, b.reshape(1, D))
    src = edge_index[0]
    dst3 = edge_index[1].reshape(N_STAGES, 1, STAGE)
    zrows = jnp.zeros((NODE_BASE + 8, D), jnp.float32)
    partials = _sc_scatter(hidden, src, dst3, edge_weight, zrows)
    return _combine(partials[0], partials[1])
